# Initial kernel scaffold; baseline (speedup 1.0000x reference)
import jax
import jax.numpy as jnp
from jax.experimental import pallas as pl


def kernel(imgs, target, params, rand_vals, contrasts):
    raise NotImplementedError("write your pallas kernel here")


# speedup vs baseline: 4.4536x; 4.4536x over previous
import functools
import math

import jax
import jax.numpy as jnp
from jax.experimental import pallas as pl

_B = 8
_NUM_IMGS = 3
_H = 96
_PATCH = 8
_LAT = 12
_SEQ = _LAT * _LAT
_LD = 5
_HID = 512
_NH = 8
_DH = _HID // _NH
_FF = 2048
_K = 8192
_NC = 4
_NUM_TO_MASK = int(math.cos(0.5 * math.pi / 2.0) * _SEQ)
_NTOK = _NUM_IMGS * _SEQ + _SEQ

_INTERPRET = False


def _ln(x, g, b):
    m = x.mean(axis=-1, keepdims=True)
    v = x.var(axis=-1, keepdims=True)
    return (x - m) / jnp.sqrt(v + 1e-5) * g + b


def _kernel_a(x_ref, r_ref, c_ref, enc_w_ref, enc_b_ref, patch_w_ref,
              patch_b_ref, cemb_ref, mask_tok_ref, ln1g_ref, ln1b_ref,
              wqkv_ref, wo_ref, ln2g_ref, ln2b_ref, w1_ref, w2_ref, cb_ref,
              cbt_ref, h2_ref, labels_ref):
    x = x_ref[0]
    r = r_ref[0]
    mask_tok = mask_tok_ref[...]
    ln1g, ln1b = ln1g_ref[...], ln1b_ref[...]
    ln2g, ln2b = ln2g_ref[...], ln2b_ref[...]

    lat = x @ enc_w_ref[...] + enc_b_ref[...]
    feats = lat @ patch_w_ref[...] + patch_b_ref[...]

    rs = jnp.transpose(r)
    col = jax.lax.broadcasted_iota(jnp.int32, (_SEQ, _SEQ), 1)
    row = jax.lax.broadcasted_iota(jnp.int32, (_SEQ, _SEQ), 0)
    less = (r < rs) | ((r == rs) & (col < row))
    rank = jnp.sum(less.astype(jnp.int32), axis=1, keepdims=True)
    is_masked = rank < _NUM_TO_MASK

    tfeat = feats[_NUM_IMGS * _SEQ:]
    masked_t = jnp.where(is_masked, mask_tok, tfeat)
    x_in = jnp.concatenate([feats[:_NUM_IMGS * _SEQ], masked_t], axis=0)

    c = c_ref[0, 0, 0].astype(jnp.int32)
    crow = jax.lax.broadcasted_iota(jnp.int32, (_NC, 1), 0)
    conehot = (crow == c).astype(jnp.float32)
    ce = jnp.sum(conehot * cemb_ref[...], axis=0, keepdims=True)

    y_all = _ln(x_in, ln1g, ln1b)
    y_c = _ln(ce, ln1g, ln1b)
    y_m = _ln(mask_tok, ln1g, ln1b)
    wkv = wqkv_ref[:, _HID:]
    kv_all = y_all @ wkv
    kv_c = y_c @ wkv
    q = y_m @ wqkv_ref[:, :_HID]

    k_all, v_all = kv_all[:, :_HID], kv_all[:, _HID:]
    k_c, v_c = kv_c[:, :_HID], kv_c[:, _HID:]

    bi = jax.lax.broadcasted_iota(jnp.int32, (_HID, _NH), 0)
    bh = jax.lax.broadcasted_iota(jnp.int32, (_HID, _NH), 1)
    blockm = (bi // _DH == bh).astype(jnp.float32)
    inv_s = 1.0 / math.sqrt(_DH)
    s_all = ((k_all * q) @ blockm) * inv_s
    s_c = ((k_c * q) @ blockm) * inv_s
    m8 = jnp.maximum(jnp.max(s_all, axis=0, keepdims=True), s_c)
    e_all = jnp.exp(s_all - m8)
    e_c = jnp.exp(s_c - m8)
    den = jnp.sum(e_all, axis=0, keepdims=True) + e_c
    w_all = e_all / den
    w_c = e_c / den

    bh2 = jax.lax.broadcasted_iota(jnp.int32, (_NH, _HID), 0)
    bj2 = jax.lax.broadcasted_iota(jnp.int32, (_NH, _HID), 1)
    blockm_t = (bj2 // _DH == bh2).astype(jnp.float32)
    wv = w_all @ blockm_t
    o = (jnp.sum(wv * v_all, axis=0, keepdims=True)
         + (w_c @ blockm_t) * v_c)

    h_row = mask_tok + o @ wo_ref[...]
    y2 = _ln(h_row, ln2g, ln2b)
    h2 = h_row + jax.nn.gelu(y2 @ w1_ref[...]) @ w2_ref[...]
    h2_ref[0] = h2

    lat_t = lat[_NUM_IMGS * _SEQ:]
    cb = cb_ref[...]
    cbt = cbt_ref[...]
    dn = (((1,), (1,)), ((), ()))
    cross = jax.lax.dot_general(lat_t, cb, dn)
    lsq = [lat_t[:, i:i + 1] * lat_t[:, i:i + 1] for i in range(_LD)]
    latsq = ((lsq[0] + lsq[4]) + lsq[2]) + (lsq[1] + lsq[3])
    csq = [cbt[i:i + 1, :] * cbt[i:i + 1, :] for i in range(_LD)]
    cbsq = ((csq[0] + csq[4]) + csq[2]) + (csq[1] + csq[3])
    d = latsq - 2.0 * cross + cbsq
    dmin = jnp.min(d, axis=1, keepdims=True)
    kio = jax.lax.broadcasted_iota(jnp.int32, (_SEQ, _K), 1)
    idx = jnp.min(jnp.where(d <= dmin, kio, _K), axis=1, keepdims=True)
    jrow = jax.lax.broadcasted_iota(jnp.int32, (_NUM_TO_MASK, _SEQ), 0)
    rank_t = jnp.transpose(rank)
    idx_t = jnp.transpose(idx)
    lab = jnp.sum(jnp.where(rank_t == jrow, idx_t, 0), axis=1, keepdims=True)
    labels_ref[0] = jnp.transpose(lab)


def _kernel_b(h2_ref, labels_ref, head_w_ref, head_b_ref, lm_ref, loss_ref):
    b = pl.program_id(0)
    hrow = h2_ref[0]
    logits = hrow @ head_w_ref[...] + head_b_ref[...]
    m = jnp.max(logits)
    lse = jnp.log(jnp.sum(jnp.exp(logits - m))) + m
    logp = logits - lse

    lab = labels_ref[0]
    lab_t = jnp.transpose(lab)
    kio = jax.lax.broadcasted_iota(jnp.int32, (_NUM_TO_MASK, _K), 1)
    picked = jnp.sum(jnp.where(kio == lab_t, logp, 0.0))
    nll_sum = -picked
    smooth = -jnp.sum(logp) / _K
    partial = 0.9 * nll_sum + 0.1 * _NUM_TO_MASK * smooth
    partial2 = jnp.reshape(partial, (1, 1))

    @pl.when(b == 0)
    def _():
        loss_ref[...] = jnp.zeros((1, 1), jnp.float32)

    loss_ref[...] += partial2

    @pl.when(b == _B - 1)
    def _():
        loss_ref[...] = loss_ref[...] / (_B * _NUM_TO_MASK)

    lm_ref[0] = jnp.broadcast_to(logits, (_NUM_TO_MASK, _K))


def _patchify(img):
    b = img.shape[0]
    x = img.reshape(b, 1, _LAT, _PATCH, _LAT, _PATCH)
    x = x.transpose(0, 2, 4, 1, 3, 5)
    return x.reshape(b, _SEQ, _PATCH * _PATCH)


@functools.partial(jax.jit, static_argnames=())
def kernel(imgs, target, params, rand_vals, contrasts):
    p = params
    x_parts = [_patchify(imgs[i]) for i in range(_NUM_IMGS)]
    x_parts.append(_patchify(target))
    x_all = jnp.concatenate(x_parts, axis=1)
    r3 = rand_vals.reshape(_B, 1, _SEQ)
    c3 = contrasts.reshape(_B, 1, 1).astype(jnp.int32)

    row2 = lambda a: a.reshape(1, -1)
    f32 = jnp.float32

    full = lambda shape: pl.BlockSpec(shape, lambda b: (0,) * len(shape))
    h2, labels = pl.pallas_call(
        _kernel_a,
        grid=(_B,),
        in_specs=[
            pl.BlockSpec((1, _NTOK, 64), lambda b: (b, 0, 0)),
            pl.BlockSpec((1, 1, _SEQ), lambda b: (b, 0, 0)),
            pl.BlockSpec((1, 1, 1), lambda b: (b, 0, 0)),
            full((64, _LD)), full((1, _LD)), full((_LD, _HID)),
            full((1, _HID)), full((_NC, _HID)), full((1, _HID)),
            full((1, _HID)), full((1, _HID)),
            full((_HID, 3 * _HID)), full((_HID, _HID)),
            full((1, _HID)), full((1, _HID)),
            full((_HID, _FF)), full((_FF, _HID)),
            full((_K, _LD)), full((_LD, _K)),
        ],
        out_specs=[
            pl.BlockSpec((1, 1, _HID), lambda b: (b, 0, 0)),
            pl.BlockSpec((1, 1, _NUM_TO_MASK), lambda b: (b, 0, 0)),
        ],
        out_shape=[
            jax.ShapeDtypeStruct((_B, 1, _HID), f32),
            jax.ShapeDtypeStruct((_B, 1, _NUM_TO_MASK), jnp.int32),
        ],
        interpret=_INTERPRET,
    )(x_all, r3, c3, p['enc_w'], row2(p['enc_b']), p['patch_w'],
      row2(p['patch_b']), p['contrast_embedding'], row2(p['mask_token']),
      row2(p['ln1_g']), row2(p['ln1_b']), p['wqkv'], p['wo'],
      row2(p['ln2_g']), row2(p['ln2_b']), p['w1'], p['w2'], p['codebook'],
      jnp.transpose(p['codebook']))

    logits_masked, loss2 = pl.pallas_call(
        _kernel_b,
        grid=(_B,),
        in_specs=[
            pl.BlockSpec((1, 1, _HID), lambda b: (b, 0, 0)),
            pl.BlockSpec((1, 1, _NUM_TO_MASK), lambda b: (b, 0, 0)),
            full((_HID, _K)), full((1, _K)),
        ],
        out_specs=[
            pl.BlockSpec((1, _NUM_TO_MASK, _K), lambda b: (b, 0, 0)),
            pl.BlockSpec((1, 1), lambda b: (0, 0)),
        ],
        out_shape=[
            jax.ShapeDtypeStruct((_B, _NUM_TO_MASK, _K), f32),
            jax.ShapeDtypeStruct((1, 1), f32),
        ],
        interpret=_INTERPRET,
    )(h2, labels, p['head_w'], row2(p['head_b']))

    loss = loss2[0, 0]
    labels_masked = labels.reshape(_B, _NUM_TO_MASK)
    return (loss, logits_masked, labels_masked)


# speedup vs baseline: 5.0399x; 1.1316x over previous
import functools
import math

import jax
import jax.numpy as jnp
from jax.experimental import pallas as pl
from jax.experimental.pallas import tpu as pltpu

_B = 8
_NUM_IMGS = 3
_H = 96
_PATCH = 8
_LAT = 12
_SEQ = _LAT * _LAT
_LD = 5
_HID = 512
_NH = 8
_DH = _HID // _NH
_FF = 2048
_K = 8192
_NC = 4
_NUM_TO_MASK = int(math.cos(0.5 * math.pi / 2.0) * _SEQ)
_NTOK = _NUM_IMGS * _SEQ + _SEQ

_INTERPRET = False


def _ln(x, g, b):
    m = x.mean(axis=-1, keepdims=True)
    v = x.var(axis=-1, keepdims=True)
    return (x - m) / jnp.sqrt(v + 1e-5) * g + b


def _kernel_a(x_ref, r_ref, c_ref, enc_w_ref, enc_b_ref, patch_w_ref,
              patch_b_ref, cemb_ref, mask_tok_ref, ln1g_ref, ln1b_ref,
              wqkv_ref, cb_ref, cbt_ref, o_ref, labels_ref):
    x = x_ref[0]
    r = r_ref[0]
    mask_tok = mask_tok_ref[...]
    ln1g, ln1b = ln1g_ref[...], ln1b_ref[...]

    lat = x @ enc_w_ref[...] + enc_b_ref[...]
    feats = lat @ patch_w_ref[...] + patch_b_ref[...]

    rs = jnp.transpose(r)
    col = jax.lax.broadcasted_iota(jnp.int32, (_SEQ, _SEQ), 1)
    row = jax.lax.broadcasted_iota(jnp.int32, (_SEQ, _SEQ), 0)
    less = (r < rs) | ((r == rs) & (col < row))
    rank = jnp.sum(less.astype(jnp.int32), axis=1, keepdims=True)
    is_masked = rank < _NUM_TO_MASK

    tfeat = feats[_NUM_IMGS * _SEQ:]
    masked_t = jnp.where(is_masked, mask_tok, tfeat)
    x_in = jnp.concatenate([feats[:_NUM_IMGS * _SEQ], masked_t], axis=0)

    c = c_ref[0, 0, 0].astype(jnp.int32)
    crow = jax.lax.broadcasted_iota(jnp.int32, (_NC, 1), 0)
    conehot = (crow == c).astype(jnp.float32)
    ce = jnp.sum(conehot * cemb_ref[...], axis=0, keepdims=True)

    y_all = _ln(x_in, ln1g, ln1b)
    y_c = _ln(ce, ln1g, ln1b)
    y_m = _ln(mask_tok, ln1g, ln1b)
    wkv = wqkv_ref[:, _HID:]
    kv_all = y_all @ wkv
    kv_c = y_c @ wkv
    q = y_m @ wqkv_ref[:, :_HID]

    k_all, v_all = kv_all[:, :_HID], kv_all[:, _HID:]
    k_c, v_c = kv_c[:, :_HID], kv_c[:, _HID:]

    bi = jax.lax.broadcasted_iota(jnp.int32, (_HID, _NH), 0)
    bh = jax.lax.broadcasted_iota(jnp.int32, (_HID, _NH), 1)
    blockm = (bi // _DH == bh).astype(jnp.float32)
    inv_s = 1.0 / math.sqrt(_DH)
    s_all = ((k_all * q) @ blockm) * inv_s
    s_c = ((k_c * q) @ blockm) * inv_s
    m8 = jnp.maximum(jnp.max(s_all, axis=0, keepdims=True), s_c)
    e_all = jnp.exp(s_all - m8)
    e_c = jnp.exp(s_c - m8)
    den = jnp.sum(e_all, axis=0, keepdims=True) + e_c
    w_all = e_all / den
    w_c = e_c / den

    bh2 = jax.lax.broadcasted_iota(jnp.int32, (_NH, _HID), 0)
    bj2 = jax.lax.broadcasted_iota(jnp.int32, (_NH, _HID), 1)
    blockm_t = (bj2 // _DH == bh2).astype(jnp.float32)
    wv = w_all @ blockm_t
    o = (jnp.sum(wv * v_all, axis=0, keepdims=True)
         + (w_c @ blockm_t) * v_c)
    o_ref[0] = o

    lat_t = lat[_NUM_IMGS * _SEQ:]
    cb = cb_ref[...]
    cbt = cbt_ref[...]
    dn = (((1,), (1,)), ((), ()))
    cross = jax.lax.dot_general(lat_t, cb, dn)
    lsq = [lat_t[:, i:i + 1] * lat_t[:, i:i + 1] for i in range(_LD)]
    latsq = ((lsq[0] + lsq[4]) + lsq[2]) + (lsq[1] + lsq[3])
    csq = [cbt[i:i + 1, :] * cbt[i:i + 1, :] for i in range(_LD)]
    cbsq = ((csq[0] + csq[4]) + csq[2]) + (csq[1] + csq[3])
    d = latsq - 2.0 * cross + cbsq
    dmin = jnp.min(d, axis=1, keepdims=True)
    kio = jax.lax.broadcasted_iota(jnp.int32, (_SEQ, _K), 1)
    idx = jnp.min(jnp.where(d <= dmin, kio, _K), axis=1, keepdims=True)
    jrow = jax.lax.broadcasted_iota(jnp.int32, (_NUM_TO_MASK, _SEQ), 0)
    rank_t = jnp.transpose(rank)
    idx_t = jnp.transpose(idx)
    lab = jnp.sum(jnp.where(rank_t == jrow, idx_t, 0), axis=1, keepdims=True)
    labels_ref[0] = jnp.transpose(lab)


def _kernel_b(o_ref, labels_ref, mask_tok_ref, wo_ref, ln2g_ref, ln2b_ref,
              w1_ref, w2_ref, head_w_ref, head_b_ref, lm_ref, loss_ref,
              logits_s):
    b = pl.program_id(0)

    @pl.when(b == 0)
    def _():
        o8 = o_ref[...].reshape(_B, _HID)
        h = mask_tok_ref[...] + o8 @ wo_ref[...]
        y2 = _ln(h, ln2g_ref[...], ln2b_ref[...])
        h2 = h + jax.nn.gelu(y2 @ w1_ref[...]) @ w2_ref[...]
        logits8 = h2 @ head_w_ref[...] + head_b_ref[...]
        logits_s[...] = logits8
        m8 = jnp.max(logits8, axis=1, keepdims=True)
        lse8 = (jnp.log(jnp.sum(jnp.exp(logits8 - m8), axis=1, keepdims=True))
                + m8)
        sum8 = jnp.sum(logits8, axis=1, keepdims=True)

        kio = jax.lax.broadcasted_iota(jnp.int32, (_NUM_TO_MASK, _K), 1)
        loss_acc = jnp.zeros((1, 1), jnp.float32)
        for bb in range(_B):
            lab_t = jnp.transpose(labels_ref[bb])
            row = logits_s[bb:bb + 1]
            picked_lg = jnp.sum(jnp.where(kio == lab_t, row, 0.0))
            picked = picked_lg - _NUM_TO_MASK * lse8[bb, 0]
            nll_sum = -picked
            smooth = -(sum8[bb, 0] - _K * lse8[bb, 0]) / _K
            partial = 0.9 * nll_sum + 0.1 * _NUM_TO_MASK * smooth
            loss_acc = loss_acc + jnp.reshape(partial, (1, 1))
        loss_ref[...] = loss_acc / (_B * _NUM_TO_MASK)

    row_b = logits_s[pl.ds(b, 1), :]
    lm_ref[0] = jnp.broadcast_to(row_b, (_NUM_TO_MASK, _K))


def _patchify(img):
    b = img.shape[0]
    x = img.reshape(b, 1, _LAT, _PATCH, _LAT, _PATCH)
    x = x.transpose(0, 2, 4, 1, 3, 5)
    return x.reshape(b, _SEQ, _PATCH * _PATCH)


@functools.partial(jax.jit, static_argnames=())
def kernel(imgs, target, params, rand_vals, contrasts):
    p = params
    x_parts = [_patchify(imgs[i]) for i in range(_NUM_IMGS)]
    x_parts.append(_patchify(target))
    x_all = jnp.concatenate(x_parts, axis=1)
    r3 = rand_vals.reshape(_B, 1, _SEQ)
    c3 = contrasts.reshape(_B, 1, 1).astype(jnp.int32)

    row2 = lambda a: a.reshape(1, -1)
    f32 = jnp.float32

    full = lambda shape: pl.BlockSpec(shape, lambda b: (0,) * len(shape))
    o_all, labels = pl.pallas_call(
        _kernel_a,
        grid=(_B,),
        in_specs=[
            pl.BlockSpec((1, _NTOK, 64), lambda b: (b, 0, 0)),
            pl.BlockSpec((1, 1, _SEQ), lambda b: (b, 0, 0)),
            pl.BlockSpec((1, 1, 1), lambda b: (b, 0, 0)),
            full((64, _LD)), full((1, _LD)), full((_LD, _HID)),
            full((1, _HID)), full((_NC, _HID)), full((1, _HID)),
            full((1, _HID)), full((1, _HID)),
            full((_HID, 3 * _HID)),
            full((_K, _LD)), full((_LD, _K)),
        ],
        out_specs=[
            pl.BlockSpec((1, 1, _HID), lambda b: (b, 0, 0)),
            pl.BlockSpec((1, 1, _NUM_TO_MASK), lambda b: (b, 0, 0)),
        ],
        out_shape=[
            jax.ShapeDtypeStruct((_B, 1, _HID), f32),
            jax.ShapeDtypeStruct((_B, 1, _NUM_TO_MASK), jnp.int32),
        ],
        interpret=_INTERPRET,
    )(x_all, r3, c3, p['enc_w'], row2(p['enc_b']), p['patch_w'],
      row2(p['patch_b']), p['contrast_embedding'], row2(p['mask_token']),
      row2(p['ln1_g']), row2(p['ln1_b']), p['wqkv'], p['codebook'],
      jnp.transpose(p['codebook']))

    logits_masked, loss2 = pl.pallas_call(
        _kernel_b,
        grid=(_B,),
        in_specs=[
            full((_B, 1, _HID)),
            full((_B, 1, _NUM_TO_MASK)),
            full((1, _HID)), full((_HID, _HID)),
            full((1, _HID)), full((1, _HID)),
            full((_HID, _FF)), full((_FF, _HID)),
            full((_HID, _K)), full((1, _K)),
        ],
        out_specs=[
            pl.BlockSpec((1, _NUM_TO_MASK, _K), lambda b: (b, 0, 0)),
            pl.BlockSpec((1, 1), lambda b: (0, 0)),
        ],
        out_shape=[
            jax.ShapeDtypeStruct((_B, _NUM_TO_MASK, _K), f32),
            jax.ShapeDtypeStruct((1, 1), f32),
        ],
        scratch_shapes=[pltpu.VMEM((_B, _K), jnp.float32)],
        interpret=_INTERPRET,
    )(o_all, labels, row2(p['mask_token']), p['wo'],
      row2(p['ln2_g']), row2(p['ln2_b']), p['w1'], p['w2'],
      p['head_w'], row2(p['head_b']))

    loss = loss2[0, 0]
    labels_masked = labels.reshape(_B, _NUM_TO_MASK)
    return (loss, logits_masked, labels_masked)


# speedup vs baseline: 6.9906x; 1.3870x over previous
import functools
import math

import jax
import jax.numpy as jnp
from jax.experimental import pallas as pl
from jax.experimental.pallas import tpu as pltpu

_B = 8
_NUM_IMGS = 3
_H = 96
_PATCH = 8
_LAT = 12
_SEQ = _LAT * _LAT
_LD = 5
_HID = 512
_NH = 8
_DH = _HID // _NH
_FF = 2048
_K = 8192
_NC = 4
_NUM_TO_MASK = int(math.cos(0.5 * math.pi / 2.0) * _SEQ)
_NTOK = _NUM_IMGS * _SEQ + _SEQ

_INTERPRET = False


def _ln(x, g, b):
    m = x.mean(axis=-1, keepdims=True)
    v = x.var(axis=-1, keepdims=True)
    return (x - m) / jnp.sqrt(v + 1e-5) * g + b


def _kernel_a(x_ref, r_ref, c_ref, enc_w_ref, enc_b_ref, patch_w_ref,
              patch_b_ref, cemb_ref, mask_tok_ref, ln1g_ref, ln1b_ref,
              wqkv_ref, cb_ref, cbt_ref, o_ref, labels_ref):
    x4 = x_ref[0]
    xs = [x4[i].reshape(_LAT, _PATCH, _LAT, _PATCH).transpose(0, 2, 1, 3)
          .reshape(_SEQ, _PATCH * _PATCH) for i in range(_NUM_IMGS + 1)]
    x = jnp.concatenate(xs, axis=0)
    r = r_ref[0]
    mask_tok = mask_tok_ref[...]
    ln1g, ln1b = ln1g_ref[...], ln1b_ref[...]

    lat = x @ enc_w_ref[...] + enc_b_ref[...]
    feats = lat @ patch_w_ref[...] + patch_b_ref[...]

    rs = jnp.transpose(r)
    col = jax.lax.broadcasted_iota(jnp.int32, (_SEQ, _SEQ), 1)
    row = jax.lax.broadcasted_iota(jnp.int32, (_SEQ, _SEQ), 0)
    less = (r < rs) | ((r == rs) & (col < row))
    rank = jnp.sum(less.astype(jnp.int32), axis=1, keepdims=True)
    is_masked = rank < _NUM_TO_MASK

    tfeat = feats[_NUM_IMGS * _SEQ:]
    masked_t = jnp.where(is_masked, mask_tok, tfeat)
    x_in = jnp.concatenate([feats[:_NUM_IMGS * _SEQ], masked_t], axis=0)

    c = c_ref[0, 0, 0].astype(jnp.int32)
    crow = jax.lax.broadcasted_iota(jnp.int32, (_NC, 1), 0)
    conehot = (crow == c).astype(jnp.float32)
    ce = jnp.sum(conehot * cemb_ref[...], axis=0, keepdims=True)

    y_all = _ln(x_in, ln1g, ln1b)
    y_c = _ln(ce, ln1g, ln1b)
    y_m = _ln(mask_tok, ln1g, ln1b)
    wkv = wqkv_ref[:, _HID:]
    kv_all = y_all @ wkv
    kv_c = y_c @ wkv
    q = y_m @ wqkv_ref[:, :_HID]

    k_all, v_all = kv_all[:, :_HID], kv_all[:, _HID:]
    k_c, v_c = kv_c[:, :_HID], kv_c[:, _HID:]

    bi = jax.lax.broadcasted_iota(jnp.int32, (_HID, _NH), 0)
    bh = jax.lax.broadcasted_iota(jnp.int32, (_HID, _NH), 1)
    blockm = (bi // _DH == bh).astype(jnp.float32)
    inv_s = 1.0 / math.sqrt(_DH)
    s_all = ((k_all * q) @ blockm) * inv_s
    s_c = ((k_c * q) @ blockm) * inv_s
    m8 = jnp.maximum(jnp.max(s_all, axis=0, keepdims=True), s_c)
    e_all = jnp.exp(s_all - m8)
    e_c = jnp.exp(s_c - m8)
    den = jnp.sum(e_all, axis=0, keepdims=True) + e_c
    w_all = e_all / den
    w_c = e_c / den

    bh2 = jax.lax.broadcasted_iota(jnp.int32, (_NH, _HID), 0)
    bj2 = jax.lax.broadcasted_iota(jnp.int32, (_NH, _HID), 1)
    blockm_t = (bj2 // _DH == bh2).astype(jnp.float32)
    wv = w_all @ blockm_t
    o = (jnp.sum(wv * v_all, axis=0, keepdims=True)
         + (w_c @ blockm_t) * v_c)
    o_ref[0] = o

    lat_t = lat[_NUM_IMGS * _SEQ:]
    cb = cb_ref[...]
    cbt = cbt_ref[...]
    dn = (((1,), (1,)), ((), ()))
    cross = jax.lax.dot_general(lat_t, cb, dn)
    lsq = [lat_t[:, i:i + 1] * lat_t[:, i:i + 1] for i in range(_LD)]
    latsq = ((lsq[0] + lsq[4]) + lsq[2]) + (lsq[1] + lsq[3])
    csq = [cbt[i:i + 1, :] * cbt[i:i + 1, :] for i in range(_LD)]
    cbsq = ((csq[0] + csq[4]) + csq[2]) + (csq[1] + csq[3])
    d = latsq - 2.0 * cross + cbsq
    dmin = jnp.min(d, axis=1, keepdims=True)
    kio = jax.lax.broadcasted_iota(jnp.int32, (_SEQ, _K), 1)
    idx = jnp.min(jnp.where(d <= dmin, kio, _K), axis=1, keepdims=True)
    jrow = jax.lax.broadcasted_iota(jnp.int32, (_NUM_TO_MASK, _SEQ), 0)
    rank_t = jnp.transpose(rank)
    idx_t = jnp.transpose(idx)
    lab = jnp.sum(jnp.where(rank_t == jrow, idx_t, 0), axis=1, keepdims=True)
    labels_ref[0] = jnp.transpose(lab)


def _kernel_b(o_ref, labels_ref, mask_tok_ref, wo_ref, ln2g_ref, ln2b_ref,
              w1_ref, w2_ref, head_w_ref, head_b_ref, lm_ref, loss_ref,
              logits_s):
    b = pl.program_id(0)

    @pl.when(b == 0)
    def _():
        o8 = o_ref[...].reshape(_B, _HID)
        h = mask_tok_ref[...] + o8 @ wo_ref[...]
        y2 = _ln(h, ln2g_ref[...], ln2b_ref[...])
        h2 = h + jax.nn.gelu(y2 @ w1_ref[...]) @ w2_ref[...]
        logits8 = h2 @ head_w_ref[...] + head_b_ref[...]
        logits_s[...] = logits8
        m8 = jnp.max(logits8, axis=1, keepdims=True)
        lse8 = (jnp.log(jnp.sum(jnp.exp(logits8 - m8), axis=1, keepdims=True))
                + m8)
        sum8 = jnp.sum(logits8, axis=1, keepdims=True)

        kio = jax.lax.broadcasted_iota(jnp.int32, (_NUM_TO_MASK, _K), 1)
        loss_acc = jnp.zeros((1, 1), jnp.float32)
        for bb in range(_B):
            lab_t = jnp.transpose(labels_ref[bb])
            row = logits_s[bb:bb + 1]
            picked_lg = jnp.sum(jnp.where(kio == lab_t, row, 0.0))
            picked = picked_lg - _NUM_TO_MASK * lse8[bb, 0]
            nll_sum = -picked
            smooth = -(sum8[bb, 0] - _K * lse8[bb, 0]) / _K
            partial = 0.9 * nll_sum + 0.1 * _NUM_TO_MASK * smooth
            loss_acc = loss_acc + jnp.reshape(partial, (1, 1))
        loss_ref[...] = loss_acc / (_B * _NUM_TO_MASK)

    row_b = logits_s[pl.ds(b, 1), :]
    lm_ref[0] = jnp.broadcast_to(row_b, (_NUM_TO_MASK, _K))


def _patchify(img):
    b = img.shape[0]
    x = img.reshape(b, 1, _LAT, _PATCH, _LAT, _PATCH)
    x = x.transpose(0, 2, 4, 1, 3, 5)
    return x.reshape(b, _SEQ, _PATCH * _PATCH)


@functools.partial(jax.jit, static_argnames=())
def kernel(imgs, target, params, rand_vals, contrasts):
    p = params
    x_all = jnp.concatenate(
        [jnp.transpose(imgs[:, :, 0], (1, 0, 2, 3)), target], axis=1
    )
    r3 = rand_vals.reshape(_B, 1, _SEQ)
    c3 = contrasts.reshape(_B, 1, 1).astype(jnp.int32)

    row2 = lambda a: a.reshape(1, -1)
    f32 = jnp.float32

    full = lambda shape: pl.BlockSpec(shape, lambda b: (0,) * len(shape))
    o_all, labels = pl.pallas_call(
        _kernel_a,
        grid=(_B,),
        in_specs=[
            pl.BlockSpec((1, _NUM_IMGS + 1, _H, _H), lambda b: (b, 0, 0, 0)),
            pl.BlockSpec((1, 1, _SEQ), lambda b: (b, 0, 0)),
            pl.BlockSpec((1, 1, 1), lambda b: (b, 0, 0)),
            full((64, _LD)), full((1, _LD)), full((_LD, _HID)),
            full((1, _HID)), full((_NC, _HID)), full((1, _HID)),
            full((1, _HID)), full((1, _HID)),
            full((_HID, 3 * _HID)),
            full((_K, _LD)), full((_LD, _K)),
        ],
        out_specs=[
            pl.BlockSpec((1, 1, _HID), lambda b: (b, 0, 0)),
            pl.BlockSpec((1, 1, _NUM_TO_MASK), lambda b: (b, 0, 0)),
        ],
        out_shape=[
            jax.ShapeDtypeStruct((_B, 1, _HID), f32),
            jax.ShapeDtypeStruct((_B, 1, _NUM_TO_MASK), jnp.int32),
        ],
        interpret=_INTERPRET,
    )(x_all, r3, c3, p['enc_w'], row2(p['enc_b']), p['patch_w'],
      row2(p['patch_b']), p['contrast_embedding'], row2(p['mask_token']),
      row2(p['ln1_g']), row2(p['ln1_b']), p['wqkv'], p['codebook'],
      jnp.transpose(p['codebook']))

    logits_masked, loss2 = pl.pallas_call(
        _kernel_b,
        grid=(_B,),
        in_specs=[
            full((_B, 1, _HID)),
            full((_B, 1, _NUM_TO_MASK)),
            full((1, _HID)), full((_HID, _HID)),
            full((1, _HID)), full((1, _HID)),
            full((_HID, _FF)), full((_FF, _HID)),
            full((_HID, _K)), full((1, _K)),
        ],
        out_specs=[
            pl.BlockSpec((1, _NUM_TO_MASK, _K), lambda b: (b, 0, 0)),
            pl.BlockSpec((1, 1), lambda b: (0, 0)),
        ],
        out_shape=[
            jax.ShapeDtypeStruct((_B, _NUM_TO_MASK, _K), f32),
            jax.ShapeDtypeStruct((1, 1), f32),
        ],
        scratch_shapes=[pltpu.VMEM((_B, _K), jnp.float32)],
        interpret=_INTERPRET,
    )(o_all, labels, row2(p['mask_token']), p['wo'],
      row2(p['ln2_g']), row2(p['ln2_b']), p['w1'], p['w2'],
      p['head_w'], row2(p['head_b']))

    loss = loss2[0, 0]
    labels_masked = labels.reshape(_B, _NUM_TO_MASK)
    return (loss, logits_masked, labels_masked)


# speedup vs baseline: 7.3439x; 1.0505x over previous
import functools
import math

import jax
import jax.numpy as jnp
from jax.experimental import pallas as pl
from jax.experimental.pallas import tpu as pltpu

_B = 8
_NUM_IMGS = 3
_H = 96
_PATCH = 8
_LAT = 12
_SEQ = _LAT * _LAT
_LD = 5
_HID = 512
_NH = 8
_DH = _HID // _NH
_FF = 2048
_K = 8192
_NC = 4
_NUM_TO_MASK = int(math.cos(0.5 * math.pi / 2.0) * _SEQ)
_NTOK = _NUM_IMGS * _SEQ + _SEQ

_INTERPRET = False


def _ln(x, g, b):
    m = x.mean(axis=-1, keepdims=True)
    v = x.var(axis=-1, keepdims=True)
    return (x - m) / jnp.sqrt(v + 1e-5) * g + b


def _patch2d(a):
    return (a.reshape(_LAT, _PATCH, _LAT, _PATCH).transpose(0, 2, 1, 3)
            .reshape(_SEQ, _PATCH * _PATCH))


def _kernel_a(imgs_ref, tgt_ref, r_ref, c_ref, enc_w_ref, enc_b_ref,
              patch_w_ref, patch_b_ref, cemb_ref, mask_tok_ref, ln1g_ref,
              ln1b_ref, wqkv_ref, cb_ref, cbt_ref, o_ref, labels_ref):
    xs = [_patch2d(imgs_ref[i, 0, 0]) for i in range(_NUM_IMGS)]
    xs.append(_patch2d(tgt_ref[0, 0]))
    x = jnp.concatenate(xs, axis=0)
    r = r_ref[0]
    mask_tok = mask_tok_ref[...]
    ln1g, ln1b = ln1g_ref[...], ln1b_ref[...]

    lat = x @ enc_w_ref[...] + enc_b_ref[...]
    feats = lat @ patch_w_ref[...] + patch_b_ref[...]

    rs = jnp.transpose(r)
    col = jax.lax.broadcasted_iota(jnp.int32, (_SEQ, _SEQ), 1)
    row = jax.lax.broadcasted_iota(jnp.int32, (_SEQ, _SEQ), 0)
    less = (r < rs) | ((r == rs) & (col < row))
    rank = jnp.sum(less.astype(jnp.int32), axis=1, keepdims=True)
    is_masked = rank < _NUM_TO_MASK

    tfeat = feats[_NUM_IMGS * _SEQ:]
    masked_t = jnp.where(is_masked, mask_tok, tfeat)
    x_in = jnp.concatenate([feats[:_NUM_IMGS * _SEQ], masked_t], axis=0)

    c = c_ref[0, 0, 0].astype(jnp.int32)
    crow = jax.lax.broadcasted_iota(jnp.int32, (_NC, 1), 0)
    conehot = (crow == c).astype(jnp.float32)
    ce = jnp.sum(conehot * cemb_ref[...], axis=0, keepdims=True)

    y_all = _ln(x_in, ln1g, ln1b)
    y_c = _ln(ce, ln1g, ln1b)
    y_m = _ln(mask_tok, ln1g, ln1b)
    q = y_m @ wqkv_ref[:, :_HID]

    bi = jax.lax.broadcasted_iota(jnp.int32, (_HID, _NH), 0)
    bh = jax.lax.broadcasted_iota(jnp.int32, (_HID, _NH), 1)
    blockm = (bi // _DH == bh).astype(jnp.float32)
    qh = blockm * jnp.transpose(q)
    wkq = wqkv_ref[:, _HID:2 * _HID] @ qh
    wv_m = wqkv_ref[:, 2 * _HID:]
    v_all = y_all @ wv_m
    v_c = y_c @ wv_m
    inv_s = 1.0 / math.sqrt(_DH)
    s_all = (y_all @ wkq) * inv_s
    s_c = (y_c @ wkq) * inv_s
    m8 = jnp.maximum(jnp.max(s_all, axis=0, keepdims=True), s_c)
    e_all = jnp.exp(s_all - m8)
    e_c = jnp.exp(s_c - m8)
    den = jnp.sum(e_all, axis=0, keepdims=True) + e_c
    w_all = e_all / den
    w_c = e_c / den

    bh2 = jax.lax.broadcasted_iota(jnp.int32, (_NH, _HID), 0)
    bj2 = jax.lax.broadcasted_iota(jnp.int32, (_NH, _HID), 1)
    blockm_t = (bj2 // _DH == bh2).astype(jnp.float32)
    o8 = jax.lax.dot_general(w_all, v_all,
                             (((0,), (0,)), ((), ())))
    o = (jnp.sum(blockm_t * o8, axis=0, keepdims=True)
         + (w_c @ blockm_t) * v_c)
    o_ref[0] = o

    lat_t = lat[_NUM_IMGS * _SEQ:]
    cb = cb_ref[...]
    cbt = cbt_ref[...]
    dn = (((1,), (1,)), ((), ()))
    cross = jax.lax.dot_general(lat_t, cb, dn)
    lsq = [lat_t[:, i:i + 1] * lat_t[:, i:i + 1] for i in range(_LD)]
    latsq = ((lsq[0] + lsq[4]) + lsq[2]) + (lsq[1] + lsq[3])
    csq = [cbt[i:i + 1, :] * cbt[i:i + 1, :] for i in range(_LD)]
    cbsq = ((csq[0] + csq[4]) + csq[2]) + (csq[1] + csq[3])
    d = latsq - 2.0 * cross + cbsq
    dmin = jnp.min(d, axis=1, keepdims=True)
    kio = jax.lax.broadcasted_iota(jnp.int32, (_SEQ, _K), 1)
    idx = jnp.min(jnp.where(d <= dmin, kio, _K), axis=1, keepdims=True)
    jrow = jax.lax.broadcasted_iota(jnp.int32, (_NUM_TO_MASK, _SEQ), 0)
    rank_t = jnp.transpose(rank)
    idx_t = jnp.transpose(idx)
    lab = jnp.sum(jnp.where(rank_t == jrow, idx_t, 0), axis=1, keepdims=True)
    labels_ref[0] = jnp.transpose(lab)


def _kernel_b(o_ref, labels_ref, mask_tok_ref, wo_ref, ln2g_ref, ln2b_ref,
              w1_ref, w2_ref, head_w_ref, head_b_ref, lm_ref, loss_ref,
              logits_s):
    b = pl.program_id(0)

    @pl.when(b == 0)
    def _():
        o8 = o_ref[...].reshape(_B, _HID)
        h = mask_tok_ref[...] + o8 @ wo_ref[...]
        y2 = _ln(h, ln2g_ref[...], ln2b_ref[...])
        h2 = h + jax.nn.gelu(y2 @ w1_ref[...]) @ w2_ref[...]
        logits8 = h2 @ head_w_ref[...] + head_b_ref[...]
        logits_s[...] = logits8
        m8 = jnp.max(logits8, axis=1, keepdims=True)
        lse8 = (jnp.log(jnp.sum(jnp.exp(logits8 - m8), axis=1, keepdims=True))
                + m8)
        sum8 = jnp.sum(logits8, axis=1, keepdims=True)

        kio = jax.lax.broadcasted_iota(jnp.int32, (_NUM_TO_MASK, _K), 1)
        loss_acc = jnp.zeros((1, 1), jnp.float32)
        for bb in range(_B):
            lab_t = jnp.transpose(labels_ref[bb])
            row = logits_s[bb:bb + 1]
            picked_lg = jnp.sum(jnp.where(kio == lab_t, row, 0.0))
            picked = picked_lg - _NUM_TO_MASK * lse8[bb, 0]
            nll_sum = -picked
            smooth = -(sum8[bb, 0] - _K * lse8[bb, 0]) / _K
            partial = 0.9 * nll_sum + 0.1 * _NUM_TO_MASK * smooth
            loss_acc = loss_acc + jnp.reshape(partial, (1, 1))
        loss_ref[...] = loss_acc / (_B * _NUM_TO_MASK)

    row_b = logits_s[pl.ds(b, 1), :]
    lm_ref[0] = jnp.broadcast_to(row_b, (_NUM_TO_MASK, _K))


def _patchify(img):
    b = img.shape[0]
    x = img.reshape(b, 1, _LAT, _PATCH, _LAT, _PATCH)
    x = x.transpose(0, 2, 4, 1, 3, 5)
    return x.reshape(b, _SEQ, _PATCH * _PATCH)


@functools.partial(jax.jit, static_argnames=())
def kernel(imgs, target, params, rand_vals, contrasts):
    p = params
    r3 = rand_vals.reshape(_B, 1, _SEQ)
    c3 = contrasts.reshape(_B, 1, 1).astype(jnp.int32)

    row2 = lambda a: a.reshape(1, -1)
    f32 = jnp.float32

    full = lambda shape: pl.BlockSpec(shape, lambda b: (0,) * len(shape))
    o_all, labels = pl.pallas_call(
        _kernel_a,
        grid=(_B,),
        in_specs=[
            pl.BlockSpec((_NUM_IMGS, 1, 1, _H, _H), lambda b: (0, b, 0, 0, 0)),
            pl.BlockSpec((1, 1, _H, _H), lambda b: (b, 0, 0, 0)),
            pl.BlockSpec((1, 1, _SEQ), lambda b: (b, 0, 0)),
            pl.BlockSpec((1, 1, 1), lambda b: (b, 0, 0)),
            full((64, _LD)), full((1, _LD)), full((_LD, _HID)),
            full((1, _HID)), full((_NC, _HID)), full((1, _HID)),
            full((1, _HID)), full((1, _HID)),
            full((_HID, 3 * _HID)),
            full((_K, _LD)), full((_LD, _K)),
        ],
        out_specs=[
            pl.BlockSpec((1, 1, _HID), lambda b: (b, 0, 0)),
            pl.BlockSpec((1, 1, _NUM_TO_MASK), lambda b: (b, 0, 0)),
        ],
        out_shape=[
            jax.ShapeDtypeStruct((_B, 1, _HID), f32),
            jax.ShapeDtypeStruct((_B, 1, _NUM_TO_MASK), jnp.int32),
        ],
        interpret=_INTERPRET,
    )(imgs, target, r3, c3, p['enc_w'], row2(p['enc_b']), p['patch_w'],
      row2(p['patch_b']), p['contrast_embedding'], row2(p['mask_token']),
      row2(p['ln1_g']), row2(p['ln1_b']), p['wqkv'], p['codebook'],
      jnp.transpose(p['codebook']))

    logits_masked, loss2 = pl.pallas_call(
        _kernel_b,
        grid=(_B,),
        in_specs=[
            full((_B, 1, _HID)),
            full((_B, 1, _NUM_TO_MASK)),
            full((1, _HID)), full((_HID, _HID)),
            full((1, _HID)), full((1, _HID)),
            full((_HID, _FF)), full((_FF, _HID)),
            full((_HID, _K)), full((1, _K)),
        ],
        out_specs=[
            pl.BlockSpec((1, _NUM_TO_MASK, _K), lambda b: (b, 0, 0)),
            pl.BlockSpec((1, 1), lambda b: (0, 0)),
        ],
        out_shape=[
            jax.ShapeDtypeStruct((_B, _NUM_TO_MASK, _K), f32),
            jax.ShapeDtypeStruct((1, 1), f32),
        ],
        scratch_shapes=[pltpu.VMEM((_B, _K), jnp.float32)],
        interpret=_INTERPRET,
    )(o_all, labels, row2(p['mask_token']), p['wo'],
      row2(p['ln2_g']), row2(p['ln2_b']), p['w1'], p['w2'],
      p['head_w'], row2(p['head_b']))

    loss = loss2[0, 0]
    labels_masked = labels.reshape(_B, _NUM_TO_MASK)
    return (loss, logits_masked, labels_masked)
